# baseline (speedup 1.0000x reference)
import functools

import jax
import jax.numpy as jnp
from jax import lax
from jax.experimental import pallas as pl
from jax.experimental.pallas import tpu as pltpu
from jax.experimental.pallas import tpu_sc as plsc

_NC = 2
_NS = 16
_LN = 16
_CW = 256
_NB = 2
_NSEG = 4


def _make_sc_kernels(NE, NV, HD):
    NT = _NC * _NS
    Et = NE // NT
    assert Et * NT == NE and Et % _CW == 0
    RW = Et // _CW
    NVP = -(-NV // (_NS * 8)) * (_NS * 8)
    NP = NVP // _NS
    mesh = plsc.VectorSubcoreMesh(
        core_axis_name="c", subcore_axis_name="s",
        num_cores=_NC, num_subcores=_NS)

    @functools.partial(
        pl.kernel,
        out_type=jax.ShapeDtypeStruct((_NC, NVP, _LN), jnp.float32),
        mesh=mesh,
        compiler_params=pltpu.CompilerParams(use_tc_tiling_on_sc=False),
        scratch_types=[
            pltpu.VMEM((RW, _CW), jnp.int32),
            pltpu.VMEM((_CW, _LN), jnp.float32),
            pltpu.VMEM_SHARED((NVP, _LN), jnp.float32),
        ],
    )
    def hist(dstr, ones_h, zeros_h, out, didx, ones_v, hacc):
        c = lax.axis_index("c")
        s = lax.axis_index("s")
        wid = c * _NS + s
        pltpu.sync_copy(dstr.at[wid], didx)
        pltpu.sync_copy(ones_h, ones_v)
        pltpu.sync_copy(zeros_h, hacc.at[pl.ds(s * NP, NP)])
        plsc.subcore_barrier()

        def body(j, carry):
            pltpu.sync_copy(ones_v, hacc.at[didx.at[j]], add=True)
            return carry

        lax.fori_loop(0, RW, body, 0)
        plsc.subcore_barrier()
        pltpu.sync_copy(hacc.at[pl.ds(s * NP, NP)], out.at[c, pl.ds(s * NP, NP)])

    HH = HD // 2
    ET = NE // _NS
    NCH = ET // _CW
    NB = _NB
    assert NCH * _CW == ET and NCH % NB == 0

    NSEG = _NSEG
    SCH = NCH // NSEG
    KG = NB // 2
    KS = NB - KG
    assert SCH % NB == 0
    NPT = NV // _NS

    @functools.partial(
        pl.kernel,
        out_type=jax.ShapeDtypeStruct((_NC, NVP, HH), jnp.float32),
        mesh=mesh,
        compiler_params=pltpu.CompilerParams(use_tc_tiling_on_sc=False),
        scratch_types=[
            pltpu.VMEM((SCH, _CW), jnp.int32),
            pltpu.VMEM((SCH, _CW), jnp.int32),
            [pltpu.VMEM((_CW, HH), jnp.float32)] * NB,
            [pltpu.SemaphoreType.DMA] * NB,
            [pltpu.SemaphoreType.DMA] * NB,
            pltpu.VMEM_SHARED((NVP, HH), jnp.float32),
            pltpu.VMEM_SHARED((NVP, HH), jnp.float32),
        ],
    )
    def spmm(m_h, srcr, dstr, zeros_h, out, sidx, didx, bufs, gsems, ssems,
             acc, table):
        c = lax.axis_index("c")
        s = lax.axis_index("s")
        pltpu.sync_copy(m_h.at[c, pl.ds(s * NPT, NPT)],
                        table.at[pl.ds(s * NPT, NPT)])
        pltpu.sync_copy(zeros_h, acc.at[pl.ds(s * NP, NP)])
        plsc.subcore_barrier()

        def seg(q, carry):
            pltpu.sync_copy(srcr.at[s, pl.ds(q * SCH, SCH)], sidx)
            pltpu.sync_copy(dstr.at[s, pl.ds(q * SCH, SCH)], didx)
            for t in range(KG):
                pltpu.make_async_copy(
                    table.at[sidx.at[t]], bufs[t], gsems[t]).start()

            def kbody(k, carry2):
                j0 = k * NB
                for b in range(NB):
                    j = j0 + b
                    pltpu.make_async_copy(
                        table.at[sidx.at[j]], bufs[b], gsems[b]).wait()
                    pltpu.async_copy(bufs[b], acc.at[didx.at[j]], ssems[b],
                                     add=True)
                    d = (b - KS) % NB

                    @pl.when(j + KG < SCH)
                    def _():
                        @pl.when(j >= KS)
                        def _():
                            pltpu.make_async_copy(
                                bufs[d], acc.at[didx.at[j - KS]],
                                ssems[d]).wait()
                        pltpu.make_async_copy(
                            table.at[sidx.at[j + KG]], bufs[d],
                            gsems[d]).start()
                return carry2

            lax.fori_loop(0, SCH // NB, kbody, 0)
            for j in range(SCH - NB, SCH):
                b = j % NB
                pltpu.make_async_copy(
                    bufs[b], acc.at[didx.at[j]], ssems[b]).wait()
            return carry

        lax.fori_loop(0, NSEG, seg, 0)
        plsc.subcore_barrier()
        pltpu.sync_copy(acc.at[pl.ds(s * NP, NP)], out.at[c, pl.ds(s * NP, NP)])

    return hist, spmm, NVP


def _split_store(mn_ref, mn, HH):
    mn_ref[0] = mn[:, :HH]
    mn_ref[1] = mn[:, HH:]


def _psum_cat(p_ref, m_ref, NV, HH):
    lo = p_ref[0, :NV, :] + m_ref[0]
    hi = p_ref[1, :NV, :] + m_ref[1]
    return jnp.concatenate([lo, hi], axis=1)


def _make_k0_body(NV, HH):
    def body(xp_ref, hp_ref, ing_ref, inb_ref, w0_ref, dinv_ref, m0_ref):
        hp = hp_ref[...][:, :NV, :]
        deg = 1.0 + jnp.sum(jnp.sum(hp, axis=2), axis=0) * (1.0 / _LN)
        dinv = lax.rsqrt(deg)[:, None]
        dinv_ref[...] = dinv
        x = xp_ref[...]
        mu = jnp.mean(x, axis=0, keepdims=True)
        var = jnp.mean((x - mu) ** 2, axis=0, keepdims=True)
        xb = (x - mu) / jnp.sqrt(var + 1e-5) * ing_ref[...] + inb_ref[...]
        m0 = dinv * jnp.dot(xb, w0_ref[...],
                            preferred_element_type=jnp.float32)
        _split_store(m0_ref, m0, HH)
    return body


def _make_kb_body(has_res, NV, HH):
    def body(p_ref, m_ref, hprev_ref, dinv_ref, b_ref, g_ref, be_ref, w_ref,
             h_ref, mn_ref):
        dinv = dinv_ref[...]
        hn = dinv * _psum_cat(p_ref, m_ref, NV, HH) + b_ref[...]
        mu = jnp.mean(hn, axis=0, keepdims=True)
        var = jnp.mean((hn - mu) ** 2, axis=0, keepdims=True)
        hb = (hn - mu) / jnp.sqrt(var + 1e-5) * g_ref[...] + be_ref[...]
        if has_res:
            hb = hb + hprev_ref[...]
        h = jnp.maximum(hb, 0.0)
        h_ref[...] = h
        mn = dinv * jnp.dot(h, w_ref[...],
                            preferred_element_type=jnp.float32)
        _split_store(mn_ref, mn, HH)
    return body


def _make_k4_body(G, NV, HH):
    def body(p_ref, m_ref, hprev_ref, dinv_ref, b_ref, g_ref, be_ref,
             batch_ref, lng_ref, lnb_ref, out_ref):
        dinv = dinv_ref[...]
        hn = dinv * _psum_cat(p_ref, m_ref, NV, HH) + b_ref[...]
        mu = jnp.mean(hn, axis=0, keepdims=True)
        var = jnp.mean((hn - mu) ** 2, axis=0, keepdims=True)
        hb = (hn - mu) / jnp.sqrt(var + 1e-5) * g_ref[...] + be_ref[...]
        h = jnp.maximum(hb + hprev_ref[...], 0.0)
        bat = jnp.broadcast_to(batch_ref[...], (G, NV))
        gid = lax.broadcasted_iota(jnp.int32, (G, NV), 0)
        oh = (bat == gid).astype(jnp.float32)
        cnt = jnp.sum(oh, axis=1, keepdims=True)
        ps = jnp.dot(oh, h, preferred_element_type=jnp.float32)
        pooled = ps / jnp.maximum(cnt, 1.0)
        pmu = jnp.mean(pooled, axis=1, keepdims=True)
        pvar = jnp.mean((pooled - pmu) ** 2, axis=1, keepdims=True)
        out_ref[...] = ((pooled - pmu) / jnp.sqrt(pvar + 1e-5) * lng_ref[...]
                        + lnb_ref[...])
    return body


def kernel(x, edge_index, batch, in_gamma, in_beta, W0, b0, g0, be0,
           W1, b1, g1, be1, W2, b2, g2, be2, W3, b3, g3, be3,
           ln_gamma, ln_beta):
    NV, F = x.shape
    NE = edge_index.shape[1]
    HD = W0.shape[1]
    G = 64
    NT = _NC * _NS
    HH = HD // 2
    NEP = -(-NE // (_NS * _CW * _NB * _NSEG)) * (_NS * _CW * _NB * _NSEG)
    RWH = NEP // NT // _CW
    NCH = NEP // _NS // _CW
    f32 = jnp.float32

    hist_fn, spmm_fn, NVP = _make_sc_kernels(NEP, NV, HD)
    NP = NVP // _NS
    assert NEP == NE or NVP > NV

    pad = NEP - NE
    i32 = jnp.int32
    src_flat = jnp.concatenate([edge_index[0], jnp.zeros((pad,), i32)])
    dst_flat = jnp.concatenate([edge_index[1], jnp.full((pad,), NV, i32)])
    src = src_flat.reshape(_NS, NCH, _CW)
    dst = dst_flat.reshape(_NS, NCH, _CW)
    dst_h = dst_flat.reshape(NT, RWH, _CW)
    zeros_h = jnp.zeros((NP, HH), f32)
    zeros16 = jnp.zeros((NP, _LN), f32)
    ones16 = jnp.ones((_CW, _LN), f32)

    hp = hist_fn(dst_h, ones16, zeros16)

    xp = jnp.pad(x, ((0, 0), (0, HD - F)))
    w0p = jnp.pad(W0, ((0, HD - F), (0, 0)))
    ing = jnp.pad(in_gamma, (0, HD - F)).reshape(1, HD)
    inb = jnp.pad(in_beta, (0, HD - F)).reshape(1, HD)

    sds = jax.ShapeDtypeStruct
    msd = sds((2, NV, HH), f32)
    tc_params = pltpu.CompilerParams(vmem_limit_bytes=100 * 1024 * 1024)
    dinv, m0 = pl.pallas_call(
        _make_k0_body(NV, HH),
        compiler_params=tc_params,
        out_shape=[sds((NV, 1), f32), msd],
    )(xp, hp, ing, inb, w0p)

    kb_first = pl.pallas_call(
        _make_kb_body(False, NV, HH),
        compiler_params=tc_params,
        out_shape=[sds((NV, HD), f32), msd],
    )
    kb_res = pl.pallas_call(
        _make_kb_body(True, NV, HH),
        compiler_params=tc_params,
        out_shape=[sds((NV, HD), f32), msd],
    )
    k4 = pl.pallas_call(
        _make_k4_body(G, NV, HH),
        compiler_params=tc_params,
        out_shape=sds((G, HD), f32),
    )

    r = lambda v: v.reshape(1, HD)

    p = spmm_fn(m0, src, dst, zeros_h)
    h1, m1 = kb_first(p, m0, m0, dinv, r(b0), r(g0), r(be0), W1)
    p = spmm_fn(m1, src, dst, zeros_h)
    h2, m2 = kb_res(p, m1, h1, dinv, r(b1), r(g1), r(be1), W2)
    p = spmm_fn(m2, src, dst, zeros_h)
    h3, m3 = kb_res(p, m2, h2, dinv, r(b2), r(g2), r(be2), W3)
    p = spmm_fn(m3, src, dst, zeros_h)
    out = k4(p, m3, h3, dinv, r(b3), r(g3), r(be3),
             batch.reshape(1, NV), ln_gamma.reshape(1, HD),
             ln_beta.reshape(1, HD))
    return out


# speedup vs baseline: 1.0769x; 1.0769x over previous
import functools

import jax
import jax.numpy as jnp
from jax import lax
from jax.experimental import pallas as pl
from jax.experimental.pallas import tpu as pltpu
from jax.experimental.pallas import tpu_sc as plsc

_NC = 2
_NS = 16
_LN = 16
_CW = 128
_NB = 4
_NSEG = 4


def _make_sc_kernels(NE, NV, HD):
    NT = _NC * _NS
    Et = NE // NT
    assert Et * NT == NE and Et % _CW == 0
    RW = Et // _CW
    NVP = -(-NV // (_NS * 8)) * (_NS * 8)
    NP = NVP // _NS
    mesh = plsc.VectorSubcoreMesh(
        core_axis_name="c", subcore_axis_name="s",
        num_cores=_NC, num_subcores=_NS)

    @functools.partial(
        pl.kernel,
        out_type=jax.ShapeDtypeStruct((_NC, NVP, _LN), jnp.float32),
        mesh=mesh,
        compiler_params=pltpu.CompilerParams(use_tc_tiling_on_sc=False),
        scratch_types=[
            pltpu.VMEM((RW, _CW), jnp.int32),
            pltpu.VMEM((_CW, _LN), jnp.float32),
            pltpu.VMEM_SHARED((NVP, _LN), jnp.float32),
        ],
    )
    def hist(dstr, ones_h, zeros_h, out, didx, ones_v, hacc):
        c = lax.axis_index("c")
        s = lax.axis_index("s")
        wid = c * _NS + s
        pltpu.sync_copy(dstr.at[wid], didx)
        pltpu.sync_copy(ones_h, ones_v)
        pltpu.sync_copy(zeros_h, hacc.at[pl.ds(s * NP, NP)])
        plsc.subcore_barrier()

        def body(j, carry):
            pltpu.sync_copy(ones_v, hacc.at[didx.at[j]], add=True)
            return carry

        lax.fori_loop(0, RW, body, 0)
        plsc.subcore_barrier()
        pltpu.sync_copy(hacc.at[pl.ds(s * NP, NP)], out.at[c, pl.ds(s * NP, NP)])

    HH = HD // 2
    ET = NE // _NS
    NCH = ET // _CW
    NB = _NB
    assert NCH * _CW == ET and NCH % NB == 0

    NSEG = _NSEG
    SCH = NCH // NSEG
    KG = NB // 2
    KS = NB - KG
    assert SCH % NB == 0
    NPT = NV // _NS

    @functools.partial(
        pl.kernel,
        out_type=jax.ShapeDtypeStruct((_NC, NVP, HH), jnp.float32),
        mesh=mesh,
        compiler_params=pltpu.CompilerParams(use_tc_tiling_on_sc=False),
        scratch_types=[
            pltpu.VMEM((SCH, _CW), jnp.int32),
            pltpu.VMEM((SCH, _CW), jnp.int32),
            [pltpu.VMEM((_CW, HH), jnp.float32)] * NB,
            [pltpu.SemaphoreType.DMA] * NB,
            [pltpu.SemaphoreType.DMA] * NB,
            pltpu.VMEM_SHARED((NVP, HH), jnp.float32),
            pltpu.VMEM_SHARED((NVP, HH), jnp.float32),
        ],
    )
    def spmm(m_h, srcr, dstr, zeros_h, out, sidx, didx, bufs, gsems, ssems,
             acc, table):
        c = lax.axis_index("c")
        s = lax.axis_index("s")
        pltpu.sync_copy(m_h.at[c, pl.ds(s * NPT, NPT)],
                        table.at[pl.ds(s * NPT, NPT)])
        pltpu.sync_copy(zeros_h, acc.at[pl.ds(s * NP, NP)])
        plsc.subcore_barrier()

        def seg(q, carry):
            pltpu.sync_copy(srcr.at[s, pl.ds(q * SCH, SCH)], sidx)
            pltpu.sync_copy(dstr.at[s, pl.ds(q * SCH, SCH)], didx)
            for t in range(KG):
                pltpu.make_async_copy(
                    table.at[sidx.at[t]], bufs[t], gsems[t]).start()

            def kbody(k, carry2):
                j0 = k * NB
                for b in range(NB):
                    j = j0 + b
                    pltpu.make_async_copy(
                        table.at[sidx.at[j]], bufs[b], gsems[b]).wait()
                    pltpu.async_copy(bufs[b], acc.at[didx.at[j]], ssems[b],
                                     add=True)
                    d = (b - KS) % NB

                    @pl.when(j + KG < SCH)
                    def _():
                        @pl.when(j >= KS)
                        def _():
                            pltpu.make_async_copy(
                                bufs[d], acc.at[didx.at[j - KS]],
                                ssems[d]).wait()
                        pltpu.make_async_copy(
                            table.at[sidx.at[j + KG]], bufs[d],
                            gsems[d]).start()
                return carry2

            lax.fori_loop(0, SCH // NB, kbody, 0)
            for j in range(SCH - NB, SCH):
                b = j % NB
                pltpu.make_async_copy(
                    bufs[b], acc.at[didx.at[j]], ssems[b]).wait()
            return carry

        lax.fori_loop(0, NSEG, seg, 0)
        plsc.subcore_barrier()
        pltpu.sync_copy(acc.at[pl.ds(s * NP, NP)], out.at[c, pl.ds(s * NP, NP)])

    return hist, spmm, NVP


def _split_store(mn_ref, mn, HH):
    mn_ref[0] = mn[:, :HH]
    mn_ref[1] = mn[:, HH:]


def _psum_cat(p_ref, m_ref, NV, HH):
    lo = p_ref[0, :NV, :] + m_ref[0]
    hi = p_ref[1, :NV, :] + m_ref[1]
    return jnp.concatenate([lo, hi], axis=1)


def _make_k0_body(NV, HH):
    def body(xp_ref, hp_ref, ing_ref, inb_ref, w0_ref, dinv_ref, m0_ref):
        hp = hp_ref[...][:, :NV, :]
        deg = 1.0 + jnp.sum(jnp.sum(hp, axis=2), axis=0) * (1.0 / _LN)
        dinv = lax.rsqrt(deg)[:, None]
        dinv_ref[...] = dinv
        x = xp_ref[...]
        mu = jnp.mean(x, axis=0, keepdims=True)
        var = jnp.mean((x - mu) ** 2, axis=0, keepdims=True)
        xb = (x - mu) / jnp.sqrt(var + 1e-5) * ing_ref[...] + inb_ref[...]
        m0 = dinv * jnp.dot(xb, w0_ref[...],
                            preferred_element_type=jnp.float32)
        _split_store(m0_ref, m0, HH)
    return body


def _make_kb_body(has_res, NV, HH):
    def body(p_ref, m_ref, hprev_ref, dinv_ref, b_ref, g_ref, be_ref, w_ref,
             h_ref, mn_ref):
        dinv = dinv_ref[...]
        hn = dinv * _psum_cat(p_ref, m_ref, NV, HH) + b_ref[...]
        mu = jnp.mean(hn, axis=0, keepdims=True)
        var = jnp.mean((hn - mu) ** 2, axis=0, keepdims=True)
        hb = (hn - mu) / jnp.sqrt(var + 1e-5) * g_ref[...] + be_ref[...]
        if has_res:
            hb = hb + hprev_ref[...]
        h = jnp.maximum(hb, 0.0)
        h_ref[...] = h
        mn = dinv * jnp.dot(h, w_ref[...],
                            preferred_element_type=jnp.float32)
        _split_store(mn_ref, mn, HH)
    return body


def _make_k4_body(G, NV, HH):
    def body(p_ref, m_ref, hprev_ref, dinv_ref, b_ref, g_ref, be_ref,
             batch_ref, lng_ref, lnb_ref, out_ref):
        dinv = dinv_ref[...]
        hn = dinv * _psum_cat(p_ref, m_ref, NV, HH) + b_ref[...]
        mu = jnp.mean(hn, axis=0, keepdims=True)
        var = jnp.mean((hn - mu) ** 2, axis=0, keepdims=True)
        hb = (hn - mu) / jnp.sqrt(var + 1e-5) * g_ref[...] + be_ref[...]
        h = jnp.maximum(hb + hprev_ref[...], 0.0)
        bat = jnp.broadcast_to(batch_ref[...], (G, NV))
        gid = lax.broadcasted_iota(jnp.int32, (G, NV), 0)
        oh = (bat == gid).astype(jnp.float32)
        cnt = jnp.sum(oh, axis=1, keepdims=True)
        ps = jnp.dot(oh, h, preferred_element_type=jnp.float32)
        pooled = ps / jnp.maximum(cnt, 1.0)
        pmu = jnp.mean(pooled, axis=1, keepdims=True)
        pvar = jnp.mean((pooled - pmu) ** 2, axis=1, keepdims=True)
        out_ref[...] = ((pooled - pmu) / jnp.sqrt(pvar + 1e-5) * lng_ref[...]
                        + lnb_ref[...])
    return body


def kernel(x, edge_index, batch, in_gamma, in_beta, W0, b0, g0, be0,
           W1, b1, g1, be1, W2, b2, g2, be2, W3, b3, g3, be3,
           ln_gamma, ln_beta):
    NV, F = x.shape
    NE = edge_index.shape[1]
    HD = W0.shape[1]
    G = 64
    NT = _NC * _NS
    HH = HD // 2
    NEP = -(-NE // (_NS * _CW * _NB * _NSEG)) * (_NS * _CW * _NB * _NSEG)
    RWH = NEP // NT // _CW
    NCH = NEP // _NS // _CW
    f32 = jnp.float32

    hist_fn, spmm_fn, NVP = _make_sc_kernels(NEP, NV, HD)
    NP = NVP // _NS
    assert NEP == NE or NVP > NV

    pad = NEP - NE
    i32 = jnp.int32
    src_flat = jnp.concatenate([edge_index[0], jnp.zeros((pad,), i32)])
    dst_flat = jnp.concatenate([edge_index[1], jnp.full((pad,), NV, i32)])
    src = src_flat.reshape(_NS, NCH, _CW)
    dst = dst_flat.reshape(_NS, NCH, _CW)
    dst_h = dst_flat.reshape(NT, RWH, _CW)
    zeros_h = jnp.zeros((NP, HH), f32)
    zeros16 = jnp.zeros((NP, _LN), f32)
    ones16 = jnp.ones((_CW, _LN), f32)

    hp = hist_fn(dst_h, ones16, zeros16)

    xp = jnp.pad(x, ((0, 0), (0, HD - F)))
    w0p = jnp.pad(W0, ((0, HD - F), (0, 0)))
    ing = jnp.pad(in_gamma, (0, HD - F)).reshape(1, HD)
    inb = jnp.pad(in_beta, (0, HD - F)).reshape(1, HD)

    sds = jax.ShapeDtypeStruct
    msd = sds((2, NV, HH), f32)
    tc_params = pltpu.CompilerParams(vmem_limit_bytes=100 * 1024 * 1024)
    dinv, m0 = pl.pallas_call(
        _make_k0_body(NV, HH),
        compiler_params=tc_params,
        out_shape=[sds((NV, 1), f32), msd],
    )(xp, hp, ing, inb, w0p)

    kb_first = pl.pallas_call(
        _make_kb_body(False, NV, HH),
        compiler_params=tc_params,
        out_shape=[sds((NV, HD), f32), msd],
    )
    kb_res = pl.pallas_call(
        _make_kb_body(True, NV, HH),
        compiler_params=tc_params,
        out_shape=[sds((NV, HD), f32), msd],
    )
    k4 = pl.pallas_call(
        _make_k4_body(G, NV, HH),
        compiler_params=tc_params,
        out_shape=sds((G, HD), f32),
    )

    r = lambda v: v.reshape(1, HD)

    p = spmm_fn(m0, src, dst, zeros_h)
    h1, m1 = kb_first(p, m0, m0, dinv, r(b0), r(g0), r(be0), W1)
    p = spmm_fn(m1, src, dst, zeros_h)
    h2, m2 = kb_res(p, m1, h1, dinv, r(b1), r(g1), r(be1), W2)
    p = spmm_fn(m2, src, dst, zeros_h)
    h3, m3 = kb_res(p, m2, h2, dinv, r(b2), r(g2), r(be2), W3)
    p = spmm_fn(m3, src, dst, zeros_h)
    out = k4(p, m3, h3, dinv, r(b3), r(g3), r(be3),
             batch.reshape(1, NV), ln_gamma.reshape(1, HD),
             ln_beta.reshape(1, HD))
    return out


# speedup vs baseline: 1.0777x; 1.0008x over previous
import functools

import jax
import jax.numpy as jnp
from jax import lax
from jax.experimental import pallas as pl
from jax.experimental.pallas import tpu as pltpu
from jax.experimental.pallas import tpu_sc as plsc

_NC = 2
_NS = 16
_LN = 16
_CW = 128
_NB = 4
_NSEG = 4


def _make_sc_kernels(NE, NV, HD):
    NT = _NC * _NS
    Et = NE // NT
    assert Et * NT == NE and Et % _CW == 0
    RW = Et // _CW
    NVP = -(-NV // (_NS * 8)) * (_NS * 8)
    NP = NVP // _NS
    mesh = plsc.VectorSubcoreMesh(
        core_axis_name="c", subcore_axis_name="s",
        num_cores=_NC, num_subcores=_NS)

    @functools.partial(
        pl.kernel,
        out_type=jax.ShapeDtypeStruct((_NC, NVP, _LN), jnp.float32),
        mesh=mesh,
        compiler_params=pltpu.CompilerParams(use_tc_tiling_on_sc=False),
        scratch_types=[
            pltpu.VMEM((RW, _CW), jnp.int32),
            pltpu.VMEM((_CW, _LN), jnp.float32),
            pltpu.VMEM_SHARED((NVP, _LN), jnp.float32),
        ],
    )
    def hist(dstr, ones_h, zeros_h, out, didx, ones_v, hacc):
        c = lax.axis_index("c")
        s = lax.axis_index("s")
        wid = c * _NS + s
        pltpu.sync_copy(dstr.at[wid], didx)
        pltpu.sync_copy(ones_h, ones_v)
        pltpu.sync_copy(zeros_h, hacc.at[pl.ds(s * NP, NP)])
        plsc.subcore_barrier()

        def body(j, carry):
            pltpu.sync_copy(ones_v, hacc.at[didx.at[j]], add=True)
            return carry

        lax.fori_loop(0, RW, body, 0)
        plsc.subcore_barrier()
        pltpu.sync_copy(hacc.at[pl.ds(s * NP, NP)], out.at[c, pl.ds(s * NP, NP)])

    HH = HD // 2
    ET = NE // _NS
    NCH = ET // _CW
    NB = _NB
    assert NCH * _CW == ET and NCH % NB == 0

    NSEG = _NSEG
    SCH = NCH // NSEG
    KG = 3
    KS = NB - KG
    assert SCH % NB == 0
    NPT = NV // _NS

    @functools.partial(
        pl.kernel,
        out_type=jax.ShapeDtypeStruct((_NC, NVP, HH), jnp.float32),
        mesh=mesh,
        compiler_params=pltpu.CompilerParams(use_tc_tiling_on_sc=False),
        scratch_types=[
            pltpu.VMEM((SCH, _CW), jnp.int32),
            pltpu.VMEM((SCH, _CW), jnp.int32),
            [pltpu.VMEM((_CW, HH), jnp.float32)] * NB,
            [pltpu.SemaphoreType.DMA] * NB,
            [pltpu.SemaphoreType.DMA] * NB,
            pltpu.VMEM_SHARED((NVP, HH), jnp.float32),
            pltpu.VMEM_SHARED((NVP, HH), jnp.float32),
        ],
    )
    def spmm(m_h, srcr, dstr, zeros_h, out, sidx, didx, bufs, gsems, ssems,
             acc, table):
        c = lax.axis_index("c")
        s = lax.axis_index("s")
        pltpu.sync_copy(m_h.at[c, pl.ds(s * NPT, NPT)],
                        table.at[pl.ds(s * NPT, NPT)])
        pltpu.sync_copy(zeros_h, acc.at[pl.ds(s * NP, NP)])
        plsc.subcore_barrier()

        def seg(q, carry):
            pltpu.sync_copy(srcr.at[s, pl.ds(q * SCH, SCH)], sidx)
            pltpu.sync_copy(dstr.at[s, pl.ds(q * SCH, SCH)], didx)
            for t in range(KG):
                pltpu.make_async_copy(
                    table.at[sidx.at[t]], bufs[t], gsems[t]).start()

            def kbody(k, carry2):
                j0 = k * NB
                for b in range(NB):
                    j = j0 + b
                    pltpu.make_async_copy(
                        table.at[sidx.at[j]], bufs[b], gsems[b]).wait()
                    pltpu.async_copy(bufs[b], acc.at[didx.at[j]], ssems[b],
                                     add=True)
                    d = (b - KS) % NB

                    @pl.when(j + KG < SCH)
                    def _():
                        @pl.when(j >= KS)
                        def _():
                            pltpu.make_async_copy(
                                bufs[d], acc.at[didx.at[j - KS]],
                                ssems[d]).wait()
                        pltpu.make_async_copy(
                            table.at[sidx.at[j + KG]], bufs[d],
                            gsems[d]).start()
                return carry2

            lax.fori_loop(0, SCH // NB, kbody, 0)
            for j in range(SCH - NB, SCH):
                b = j % NB
                pltpu.make_async_copy(
                    bufs[b], acc.at[didx.at[j]], ssems[b]).wait()
            return carry

        lax.fori_loop(0, NSEG, seg, 0)
        plsc.subcore_barrier()
        pltpu.sync_copy(acc.at[pl.ds(s * NP, NP)], out.at[c, pl.ds(s * NP, NP)])

    return hist, spmm, NVP


def _split_store(mn_ref, mn, HH):
    mn_ref[0] = mn[:, :HH]
    mn_ref[1] = mn[:, HH:]


def _psum_cat(p_ref, m_ref, NV, HH):
    lo = p_ref[0, :NV, :] + m_ref[0]
    hi = p_ref[1, :NV, :] + m_ref[1]
    return jnp.concatenate([lo, hi], axis=1)


def _make_k0_body(NV, HH):
    def body(xp_ref, hp_ref, ing_ref, inb_ref, w0_ref, dinv_ref, m0_ref):
        hp = hp_ref[...][:, :NV, :]
        deg = 1.0 + jnp.sum(jnp.sum(hp, axis=2), axis=0) * (1.0 / _LN)
        dinv = lax.rsqrt(deg)[:, None]
        dinv_ref[...] = dinv
        x = xp_ref[...]
        mu = jnp.mean(x, axis=0, keepdims=True)
        var = jnp.mean((x - mu) ** 2, axis=0, keepdims=True)
        xb = (x - mu) / jnp.sqrt(var + 1e-5) * ing_ref[...] + inb_ref[...]
        m0 = dinv * jnp.dot(xb, w0_ref[...],
                            preferred_element_type=jnp.float32)
        _split_store(m0_ref, m0, HH)
    return body


def _make_kb_body(has_res, NV, HH):
    def body(p_ref, m_ref, hprev_ref, dinv_ref, b_ref, g_ref, be_ref, w_ref,
             h_ref, mn_ref):
        dinv = dinv_ref[...]
        hn = dinv * _psum_cat(p_ref, m_ref, NV, HH) + b_ref[...]
        mu = jnp.mean(hn, axis=0, keepdims=True)
        var = jnp.mean((hn - mu) ** 2, axis=0, keepdims=True)
        hb = (hn - mu) / jnp.sqrt(var + 1e-5) * g_ref[...] + be_ref[...]
        if has_res:
            hb = hb + hprev_ref[...]
        h = jnp.maximum(hb, 0.0)
        h_ref[...] = h
        mn = dinv * jnp.dot(h, w_ref[...],
                            preferred_element_type=jnp.float32)
        _split_store(mn_ref, mn, HH)
    return body


def _make_k4_body(G, NV, HH):
    def body(p_ref, m_ref, hprev_ref, dinv_ref, b_ref, g_ref, be_ref,
             batch_ref, lng_ref, lnb_ref, out_ref):
        dinv = dinv_ref[...]
        hn = dinv * _psum_cat(p_ref, m_ref, NV, HH) + b_ref[...]
        mu = jnp.mean(hn, axis=0, keepdims=True)
        var = jnp.mean((hn - mu) ** 2, axis=0, keepdims=True)
        hb = (hn - mu) / jnp.sqrt(var + 1e-5) * g_ref[...] + be_ref[...]
        h = jnp.maximum(hb + hprev_ref[...], 0.0)
        bat = jnp.broadcast_to(batch_ref[...], (G, NV))
        gid = lax.broadcasted_iota(jnp.int32, (G, NV), 0)
        oh = (bat == gid).astype(jnp.float32)
        cnt = jnp.sum(oh, axis=1, keepdims=True)
        ps = jnp.dot(oh, h, preferred_element_type=jnp.float32)
        pooled = ps / jnp.maximum(cnt, 1.0)
        pmu = jnp.mean(pooled, axis=1, keepdims=True)
        pvar = jnp.mean((pooled - pmu) ** 2, axis=1, keepdims=True)
        out_ref[...] = ((pooled - pmu) / jnp.sqrt(pvar + 1e-5) * lng_ref[...]
                        + lnb_ref[...])
    return body


def kernel(x, edge_index, batch, in_gamma, in_beta, W0, b0, g0, be0,
           W1, b1, g1, be1, W2, b2, g2, be2, W3, b3, g3, be3,
           ln_gamma, ln_beta):
    NV, F = x.shape
    NE = edge_index.shape[1]
    HD = W0.shape[1]
    G = 64
    NT = _NC * _NS
    HH = HD // 2
    NEP = -(-NE // (_NS * _CW * _NB * _NSEG)) * (_NS * _CW * _NB * _NSEG)
    RWH = NEP // NT // _CW
    NCH = NEP // _NS // _CW
    f32 = jnp.float32

    hist_fn, spmm_fn, NVP = _make_sc_kernels(NEP, NV, HD)
    NP = NVP // _NS
    assert NEP == NE or NVP > NV

    pad = NEP - NE
    i32 = jnp.int32
    src_flat = jnp.concatenate([edge_index[0], jnp.zeros((pad,), i32)])
    dst_flat = jnp.concatenate([edge_index[1], jnp.full((pad,), NV, i32)])
    src = src_flat.reshape(_NS, NCH, _CW)
    dst = dst_flat.reshape(_NS, NCH, _CW)
    dst_h = dst_flat.reshape(NT, RWH, _CW)
    zeros_h = jnp.zeros((NP, HH), f32)
    zeros16 = jnp.zeros((NP, _LN), f32)
    ones16 = jnp.ones((_CW, _LN), f32)

    hp = hist_fn(dst_h, ones16, zeros16)

    xp = jnp.pad(x, ((0, 0), (0, HD - F)))
    w0p = jnp.pad(W0, ((0, HD - F), (0, 0)))
    ing = jnp.pad(in_gamma, (0, HD - F)).reshape(1, HD)
    inb = jnp.pad(in_beta, (0, HD - F)).reshape(1, HD)

    sds = jax.ShapeDtypeStruct
    msd = sds((2, NV, HH), f32)
    tc_params = pltpu.CompilerParams(vmem_limit_bytes=100 * 1024 * 1024)
    dinv, m0 = pl.pallas_call(
        _make_k0_body(NV, HH),
        compiler_params=tc_params,
        out_shape=[sds((NV, 1), f32), msd],
    )(xp, hp, ing, inb, w0p)

    kb_first = pl.pallas_call(
        _make_kb_body(False, NV, HH),
        compiler_params=tc_params,
        out_shape=[sds((NV, HD), f32), msd],
    )
    kb_res = pl.pallas_call(
        _make_kb_body(True, NV, HH),
        compiler_params=tc_params,
        out_shape=[sds((NV, HD), f32), msd],
    )
    k4 = pl.pallas_call(
        _make_k4_body(G, NV, HH),
        compiler_params=tc_params,
        out_shape=sds((G, HD), f32),
    )

    r = lambda v: v.reshape(1, HD)

    p = spmm_fn(m0, src, dst, zeros_h)
    h1, m1 = kb_first(p, m0, m0, dinv, r(b0), r(g0), r(be0), W1)
    p = spmm_fn(m1, src, dst, zeros_h)
    h2, m2 = kb_res(p, m1, h1, dinv, r(b1), r(g1), r(be1), W2)
    p = spmm_fn(m2, src, dst, zeros_h)
    h3, m3 = kb_res(p, m2, h2, dinv, r(b2), r(g2), r(be2), W3)
    p = spmm_fn(m3, src, dst, zeros_h)
    out = k4(p, m3, h3, dinv, r(b3), r(g3), r(be3),
             batch.reshape(1, NV), ln_gamma.reshape(1, HD),
             ln_beta.reshape(1, HD))
    return out
